# Initial kernel scaffold; baseline (speedup 1.0000x reference)
import jax
import jax.numpy as jnp
from jax.experimental import pallas as pl


def kernel(flat, cu_seqlens):
    raise NotImplementedError("write your pallas kernel here")


# speedup vs baseline: 4.2551x; 4.2551x over previous
import functools

import jax
import jax.numpy as jnp
from jax import lax
from jax.experimental import pallas as pl
from jax.experimental.pallas import tpu as pltpu
from jax.experimental.pallas import tpu_sc as plsc

B = 16
MAX_LEN = 2048
D = 512
TOTAL = 16384

L = 16
T = 64
NW = 32
ROWS_PER_W = (B * MAX_LEN) // NW
NTILES = ROWS_PER_W // T
WPS = MAX_LEN // ROWS_PER_W


def _zero_rows(ref, lo, hi):
    def body(j, carry):
        for c in range(D // L):
            ref[j, pl.ds(c * L, L)] = jnp.zeros((L,), jnp.float32)
        return carry
    lax.fori_loop(lo, hi, body, 0)


@functools.partial(
    pl.kernel,
    out_type=jax.ShapeDtypeStruct((B * MAX_LEN, D), jnp.float32),
    mesh=plsc.VectorSubcoreMesh(core_axis_name="c", subcore_axis_name="s"),
    scratch_types=[
        pltpu.VMEM((32,), jnp.int32),
        pltpu.VMEM((T,), jnp.int32),
        pltpu.VMEM((T, D), jnp.float32),
        pltpu.VMEM((T, D), jnp.float32),
    ],
)
def _split_sc(cu_hbm, flat_hbm, out_hbm, cu_v, idx_v, buf, zbuf):
    wid = lax.axis_index("s") * 2 + lax.axis_index("c")
    seg = wid // WPS
    r0 = (wid % WPS) * ROWS_PER_W
    out_base = wid * ROWS_PER_W

    pltpu.sync_copy(cu_hbm, cu_v)
    cu_pair = cu_v[pl.ds(seg, L)]
    cu_b = cu_pair[0]
    cu_b1 = cu_pair[1]
    seg_len = jnp.minimum(cu_b1 - cu_b, MAX_LEN)
    valid = jnp.clip(seg_len - r0, 0, ROWS_PER_W)
    nfull = valid // T
    rem = valid % T
    nvalid = nfull + jnp.where(rem > 0, 1, 0)

    _zero_rows(zbuf, 0, T)

    src0 = cu_b + r0
    for k in range(NTILES):
        @pl.when(k < nvalid)
        def _():
            for c in range(T // L):
                lane = src0 + k * T + c * L + lax.iota(jnp.int32, L)
                idx_v[pl.ds(c * L, L)] = jnp.minimum(lane, TOTAL - 1)
            pltpu.sync_copy(flat_hbm.at[idx_v], buf)

            @pl.when((k == nfull) & (rem > 0))
            def _():
                _zero_rows(buf, rem, T)

            pltpu.sync_copy(buf, out_hbm.at[pl.ds(out_base + k * T, T)])

        @pl.when(k >= nvalid)
        def _():
            pltpu.sync_copy(zbuf, out_hbm.at[pl.ds(out_base + k * T, T)])


def kernel(flat, cu_seqlens):
    cu_pad = jnp.zeros((32,), jnp.int32).at[:B + 1].set(cu_seqlens)
    out = _split_sc(cu_pad, flat)
    return out.reshape(B, MAX_LEN, D)


# speedup vs baseline: 5.1502x; 1.2104x over previous
import functools

import jax
import jax.numpy as jnp
from jax import lax
from jax.experimental import pallas as pl
from jax.experimental.pallas import tpu as pltpu
from jax.experimental.pallas import tpu_sc as plsc

B = 16
MAX_LEN = 2048
D = 512
TOTAL = 16384

L = 16
T = 64
NW = 32
ROWS_PER_W = (B * MAX_LEN) // NW
NTILES = ROWS_PER_W // T
WPS = MAX_LEN // ROWS_PER_W


def _zero_rows(ref, lo, hi):
    def body(j, carry):
        for c in range(D // L):
            ref[j, pl.ds(c * L, L)] = jnp.zeros((L,), jnp.float32)
        return carry
    lax.fori_loop(lo, hi, body, 0)


@functools.partial(
    pl.kernel,
    out_type=jax.ShapeDtypeStruct((B * MAX_LEN, D), jnp.float32),
    mesh=plsc.VectorSubcoreMesh(core_axis_name="c", subcore_axis_name="s"),
    scratch_types=[
        pltpu.VMEM((32,), jnp.int32),
        pltpu.VMEM((T,), jnp.int32),
        pltpu.VMEM((T, D), jnp.float32),
        pltpu.VMEM((T, D), jnp.float32),
        pltpu.VMEM((T, D), jnp.float32),
        pltpu.SemaphoreType.DMA,
        pltpu.SemaphoreType.DMA,
        pltpu.SemaphoreType.DMA,
    ],
)
def _split_sc(cu_hbm, flat_hbm, out_hbm, cu_v, idx_v, buf0, buf1, zbuf,
              wsem0, wsem1, zsem):
    wid = lax.axis_index("s") * 2 + lax.axis_index("c")
    seg = wid // WPS
    r0 = (wid % WPS) * ROWS_PER_W
    out_base = wid * ROWS_PER_W

    pltpu.sync_copy(cu_hbm, cu_v)
    cu_pair = cu_v[pl.ds(seg, L)]
    cu_b = cu_pair[0]
    cu_b1 = cu_pair[1]
    seg_len = jnp.minimum(cu_b1 - cu_b, MAX_LEN)
    valid = jnp.clip(seg_len - r0, 0, ROWS_PER_W)
    nfull = valid // T
    rem = valid % T
    nvalid = nfull + jnp.where(rem > 0, 1, 0)

    _zero_rows(zbuf, 0, T)

    src0 = cu_b + r0
    bufs = (buf0, buf1)
    wsems = (wsem0, wsem1)
    for k in range(NTILES):
        buf = bufs[k % 2]
        wsem = wsems[k % 2]

        @pl.when(k < nvalid)
        def _(k=k, buf=buf, wsem=wsem):
            if k >= 2:
                pltpu.make_async_copy(
                    buf, out_hbm.at[pl.ds(out_base + (k - 2) * T, T)], wsem
                ).wait()
            for c in range(T // L):
                lane = src0 + k * T + c * L + lax.iota(jnp.int32, L)
                idx_v[pl.ds(c * L, L)] = jnp.minimum(lane, TOTAL - 1)
            pltpu.sync_copy(flat_hbm.at[idx_v], buf)

            @pl.when((k == nfull) & (rem > 0))
            def _():
                _zero_rows(buf, rem, T)

            pltpu.async_copy(buf, out_hbm.at[pl.ds(out_base + k * T, T)], wsem)

        @pl.when(k >= nvalid)
        def _(k=k):
            pltpu.async_copy(zbuf, out_hbm.at[pl.ds(out_base + k * T, T)], zsem)

    for k in range(NTILES):
        @pl.when((k < nvalid) & (k + 2 >= nvalid))
        def _(k=k, buf=bufs[k % 2], wsem=wsems[k % 2]):
            pltpu.make_async_copy(
                buf, out_hbm.at[pl.ds(out_base + k * T, T)], wsem
            ).wait()

        @pl.when(k >= nvalid)
        def _(k=k):
            pltpu.make_async_copy(
                zbuf, out_hbm.at[pl.ds(out_base + k * T, T)], zsem
            ).wait()


def kernel(flat, cu_seqlens):
    cu_pad = jnp.zeros((32,), jnp.int32).at[:B + 1].set(cu_seqlens)
    out = _split_sc(cu_pad, flat)
    return out.reshape(B, MAX_LEN, D)
